# baseline (speedup 1.0000x reference)
import functools

import jax
import jax.numpy as jnp
from jax import lax
from jax.experimental import pallas as pl
from jax.experimental.pallas import tpu as pltpu
from jax.experimental.pallas import tpu_sc as plsc


@functools.cache
def _build_sc_gather(b, l, l_pad, vocab, d):
    info = plsc.get_sparse_core_info()
    nc, ns = info.num_cores, info.num_subcores
    nw = nc * ns
    assert b % nw == 0
    rows_per_w = b // nw
    assert rows_per_w % 2 == 0 and rows_per_w >= 4 and rows_per_w % 8 == 0
    assert l <= 128 and l_pad % 8 == 0

    mesh = plsc.VectorSubcoreMesh(core_axis_name="c", subcore_axis_name="s")

    @functools.partial(
        pl.kernel,
        mesh=mesh,
        out_type=jax.ShapeDtypeStruct((b, l_pad, d), jnp.float32),
        scratch_types=[
            pltpu.VMEM((rows_per_w, l_pad), jnp.int32),
            pltpu.VMEM((l_pad, d), jnp.float32),
            pltpu.VMEM((l_pad, d), jnp.float32),
            pltpu.SemaphoreType.DMA,
            pltpu.SemaphoreType.DMA,
            pltpu.SemaphoreType.DMA,
            pltpu.SemaphoreType.DMA,
        ],
    )
    def sc_gather(ids_hbm, table_hbm, emb_out,
                  idx_v, buf_a, buf_b, gsa, gsb, ssa, ssb):
        wid = lax.axis_index("s") * nc + lax.axis_index("c")
        base = wid * rows_per_w

        pltpu.sync_copy(ids_hbm.at[pl.ds(base, rows_per_w)], idx_v)

        def gather(j, buf, sem):
            return pltpu.make_async_copy(table_hbm.at[idx_v.at[j]], buf, sem)

        def scatter(j, buf, sem):
            return pltpu.make_async_copy(buf, emb_out.at[base + j], sem)

        gather(0, buf_a, gsa).start()
        gather(0, buf_a, gsa).wait()
        gather(1, buf_b, gsb).start()
        scatter(0, buf_a, ssa).start()

        def pipe(i, carry):
            c = 2 * i + 1
            gather(c, buf_b, gsb).wait()
            scatter(c - 1, buf_a, ssa).wait()
            gather(c + 1, buf_a, gsa).start()
            scatter(c, buf_b, ssb).start()
            gather(c + 1, buf_a, gsa).wait()
            scatter(c, buf_b, ssb).wait()
            gather(c + 2, buf_b, gsb).start()
            scatter(c + 1, buf_a, ssa).start()
            return carry

        lax.fori_loop(0, rows_per_w // 2 - 1, pipe, 0)

        last = rows_per_w - 1
        gather(last, buf_b, gsb).wait()
        scatter(last - 1, buf_a, ssa).wait()
        scatter(last, buf_b, ssb).start()
        scatter(last, buf_b, ssb).wait()

    return sc_gather


@functools.cache
def _build_ts(b, l):
    inv = 1.0 / float(l - 1)

    def ts_body(out_ref):
        pos = lax.broadcasted_iota(jnp.int32, (b, l), 1)
        out_ref[...] = pos.astype(jnp.float32) * inv

    return pl.pallas_call(
        ts_body, out_shape=jax.ShapeDtypeStruct((b, l), jnp.float32)
    )


def kernel(input_ids, table):
    b, l = input_ids.shape
    vocab, d = table.shape
    l_pad = (l + 7) // 8 * 8
    ids = input_ids.astype(jnp.int32)
    ids_pad = jnp.pad(ids, ((0, 0), (0, l_pad - l)))
    emb = _build_sc_gather(b, l, l_pad, vocab, d)(ids_pad, table)[:, :l, :]
    ts = _build_ts(b, l)()
    return emb, ts
